# Initial kernel scaffold; baseline (speedup 1.0000x reference)
import functools

import jax
import jax.numpy as jnp
from jax import lax
from jax.experimental import pallas as pl
from jax.experimental.pallas import tpu as pltpu
from jax.experimental.pallas import tpu_sc as plsc

_N = 10000
_E = 320000
_D = 128
_NC = 2
_NS = 16
_NW = _NC * _NS
_EPW = _E // _NW
_CB = 80
_KC = _EPW // _CB
_RPT = _N // _NS
_BR = 1000
_GRID = _N // _BR

_mesh = plsc.VectorSubcoreMesh(
    core_axis_name="c", subcore_axis_name="s",
    num_cores=_NC, num_subcores=_NS,
)


@functools.partial(
    pl.kernel,
    out_type=jax.ShapeDtypeStruct((_NC, _N), jnp.float32),
    mesh=_mesh,
    scratch_types=[
        pltpu.VMEM((_KC, _CB), jnp.int32),
        pltpu.VMEM((_CB,), jnp.float32),
        pltpu.VMEM_SHARED((_N,), jnp.float32),
    ],
)
def _sc_degree(dst_hbm, zn_hbm, out_hbm, dst_v, ones_v, cnt_sh):
    cid = lax.axis_index("c")
    sid = lax.axis_index("s")
    wid = sid * _NC + cid
    pltpu.sync_copy(dst_hbm.at[wid], dst_v)

    def fill(i, carry):
        ones_v[pl.ds(i * 16, 16)] = jnp.ones((16,), jnp.float32)
        return carry

    lax.fori_loop(0, _CB // 16, fill, 0)

    @pl.when(sid == 0)
    def _():
        pltpu.sync_copy(zn_hbm, cnt_sh)

    plsc.subcore_barrier()

    def chunk(j, carry):
        pltpu.sync_copy(ones_v, cnt_sh.at[dst_v.at[j]], add=True)
        return carry

    lax.fori_loop(0, _KC, chunk, 0)
    plsc.subcore_barrier()

    @pl.when(sid == 0)
    def _():
        pltpu.sync_copy(cnt_sh, out_hbm.at[cid])


@functools.partial(
    pl.kernel,
    out_type=jax.ShapeDtypeStruct((_NC, _N, _D), jnp.float32),
    mesh=_mesh,
    scratch_types=[
        pltpu.VMEM((_KC, _CB), jnp.int32),
        pltpu.VMEM((_KC, _CB), jnp.int32),
        pltpu.VMEM((2, _CB, _D), jnp.float32),
        pltpu.VMEM_SHARED((_N, _D), jnp.float32),
        pltpu.SemaphoreType.DMA,
        pltpu.SemaphoreType.DMA,
    ],
)
def _sc_scatter(g_hbm, src_hbm, dst_hbm, z2_hbm, out_hbm,
                src_v, dst_v, rows_v, acc_sh, sem0, sem1):
    cid = lax.axis_index("c")
    sid = lax.axis_index("s")
    wid = sid * _NC + cid
    pltpu.sync_copy(src_hbm.at[wid], src_v)
    pltpu.sync_copy(dst_hbm.at[wid], dst_v)
    r0 = sid * _RPT
    pltpu.sync_copy(z2_hbm.at[pl.ds(r0, _RPT)], acc_sh.at[pl.ds(r0, _RPT)])
    plsc.subcore_barrier()

    pltpu.async_copy(g_hbm.at[src_v.at[0]], rows_v.at[0], sem0)

    def step(j2, carry):
        a = 2 * j2
        pltpu.make_async_copy(g_hbm.at[src_v.at[a]], rows_v.at[0], sem0).wait()
        pltpu.async_copy(g_hbm.at[src_v.at[a + 1]], rows_v.at[1], sem1)
        pltpu.sync_copy(rows_v.at[0], acc_sh.at[dst_v.at[a]], add=True)
        pltpu.make_async_copy(g_hbm.at[src_v.at[a + 1]], rows_v.at[1], sem1).wait()
        pltpu.async_copy(g_hbm.at[src_v.at[a + 2]], rows_v.at[0], sem0)
        pltpu.sync_copy(rows_v.at[1], acc_sh.at[dst_v.at[a + 1]], add=True)
        return carry

    lax.fori_loop(0, (_KC - 1) // 2, step, 0)
    a_last = _KC - 1
    pltpu.make_async_copy(g_hbm.at[src_v.at[a_last]], rows_v.at[0], sem0).wait()
    pltpu.sync_copy(rows_v.at[0], acc_sh.at[dst_v.at[a_last]], add=True)

    plsc.subcore_barrier()
    pltpu.sync_copy(acc_sh.at[pl.ds(r0, _RPT)], out_hbm.at[cid, pl.ds(r0, _RPT)])


def _mm_body(x_ref, w_ref, cnt_ref, g_ref):
    h = jnp.dot(x_ref[...], w_ref[...],
                preferred_element_type=jnp.float32,
                precision=lax.Precision.HIGHEST)
    c = cnt_ref[...]
    dinv = lax.rsqrt(c[0] + c[1] + 1.0)
    g_ref[...] = h * dinv


_tc_g = pl.pallas_call(
    _mm_body,
    grid=(_GRID,),
    in_specs=[
        pl.BlockSpec((_BR, _D), lambda i: (i, 0)),
        pl.BlockSpec((_D, _D), lambda i: (0, 0)),
        pl.BlockSpec((_NC, _BR, 1), lambda i: (0, i, 0)),
    ],
    out_specs=pl.BlockSpec((_BR, _D), lambda i: (i, 0)),
    out_shape=jax.ShapeDtypeStruct((_N, _D), jnp.float32),
)


def _ep_body(p_ref, g_ref, cnt_ref, b_ref, o_ref):
    s = p_ref[0] + p_ref[1] + g_ref[...]
    c = cnt_ref[...]
    dinv = lax.rsqrt(c[0] + c[1] + 1.0)
    u = s * dinv + b_ref[...]
    o_ref[...] = jax.nn.gelu(u, approximate=False)


_tc_ep = pl.pallas_call(
    _ep_body,
    grid=(_GRID,),
    in_specs=[
        pl.BlockSpec((_NC, _BR, _D), lambda i: (0, i, 0)),
        pl.BlockSpec((_BR, _D), lambda i: (i, 0)),
        pl.BlockSpec((_NC, _BR, 1), lambda i: (0, i, 0)),
        pl.BlockSpec((1, _D), lambda i: (0, 0)),
    ],
    out_specs=pl.BlockSpec((_BR, _D), lambda i: (i, 0)),
    out_shape=jax.ShapeDtypeStruct((_N, _D), jnp.float32),
)


def kernel(x, edge_index, W, b):
    src = edge_index[0].reshape(_NW, _KC, _CB)
    dst = edge_index[1].reshape(_NW, _KC, _CB)
    zn = jnp.zeros((_N,), jnp.float32)
    z2 = jnp.zeros((_N, _D), jnp.float32)
    cnt = _sc_degree(dst, zn)
    cnt3 = cnt.reshape(_NC, _N, 1)
    g = _tc_g(x, W, cnt3)
    p = _sc_scatter(g, src, dst, z2)
    return _tc_ep(p, g, cnt3, b.reshape(1, _D))


# speedup vs baseline: 23.9014x; 23.9014x over previous
import functools

import jax
import jax.numpy as jnp
from jax import lax
from jax.experimental import pallas as pl
from jax.experimental.pallas import tpu as pltpu
from jax.experimental.pallas import tpu_sc as plsc

_N = 10000
_E = 320000
_D = 128
_DH = _D // 2
_NC = 2
_NS = 16
_NW = _NC * _NS
_EPW = _E // _NW
_CB = 80
_KC = _EPW // _CB
_EPT = _E // _NS
_KC2 = _EPT // _CB
_RPT = 624
_TAIL = _N - _NS * _RPT
_BR = 1000
_GRID = _N // _BR

_mesh = plsc.VectorSubcoreMesh(
    core_axis_name="c", subcore_axis_name="s",
    num_cores=_NC, num_subcores=_NS,
)

_sc_params = pltpu.CompilerParams(use_tc_tiling_on_sc=False)


@functools.partial(
    pl.kernel,
    out_type=jax.ShapeDtypeStruct((_NC, _N), jnp.float32),
    mesh=_mesh,
    scratch_types=[
        pltpu.VMEM((_KC, _CB), jnp.int32),
        pltpu.VMEM((_CB,), jnp.float32),
        pltpu.VMEM_SHARED((_N,), jnp.float32),
    ],
    compiler_params=_sc_params,
)
def _sc_degree(dst_hbm, zn_hbm, out_hbm, dst_v, ones_v, cnt_sh):
    cid = lax.axis_index("c")
    sid = lax.axis_index("s")
    wid = sid * _NC + cid
    pltpu.sync_copy(dst_hbm.at[wid], dst_v)

    def fill(i, carry):
        ones_v[pl.ds(i * 16, 16)] = jnp.ones((16,), jnp.float32)
        return carry

    lax.fori_loop(0, _CB // 16, fill, 0)

    @pl.when(sid == 0)
    def _():
        pltpu.sync_copy(zn_hbm, cnt_sh)

    plsc.subcore_barrier()

    def chunk(j, carry):
        pltpu.sync_copy(ones_v, cnt_sh.at[dst_v.at[j]], add=True)
        return carry

    lax.fori_loop(0, _KC, chunk, 0)
    plsc.subcore_barrier()

    @pl.when(sid == 0)
    def _():
        pltpu.sync_copy(cnt_sh, out_hbm.at[cid])


@functools.partial(
    pl.kernel,
    out_type=jax.ShapeDtypeStruct((_NC, _N, _DH), jnp.float32),
    mesh=_mesh,
    scratch_types=[
        pltpu.VMEM((_KC2, _CB), jnp.int32),
        pltpu.VMEM((_KC2, _CB), jnp.int32),
        pltpu.VMEM((2, _CB, _DH), jnp.float32),
        pltpu.VMEM_SHARED((_N, _DH), jnp.float32),
        pltpu.SemaphoreType.DMA,
        pltpu.SemaphoreType.DMA,
    ],
    compiler_params=_sc_params,
)
def _sc_scatter(g0_hbm, g1_hbm, src_hbm, dst_hbm, z2_hbm, out_hbm,
                src_v, dst_v, rows_v, acc_sh, sem0, sem1):
    cid = lax.axis_index("c")
    sid = lax.axis_index("s")
    pltpu.sync_copy(src_hbm.at[sid], src_v)
    pltpu.sync_copy(dst_hbm.at[sid], dst_v)
    r0 = sid * _RPT
    pltpu.sync_copy(z2_hbm.at[pl.ds(r0, _RPT)], acc_sh.at[pl.ds(r0, _RPT)])

    @pl.when(sid == 0)
    def _():
        pltpu.sync_copy(z2_hbm.at[pl.ds(_NS * _RPT, _TAIL)],
                        acc_sh.at[pl.ds(_NS * _RPT, _TAIL)])

    plsc.subcore_barrier()

    def run_half(g_hbm):
        pltpu.async_copy(g_hbm.at[src_v.at[0]], rows_v.at[0], sem0)

        def step(j2, carry):
            a = 2 * j2
            pltpu.make_async_copy(g_hbm.at[src_v.at[a]],
                                  rows_v.at[0], sem0).wait()
            pltpu.async_copy(g_hbm.at[src_v.at[a + 1]], rows_v.at[1], sem1)
            pltpu.sync_copy(rows_v.at[0], acc_sh.at[dst_v.at[a]], add=True)
            pltpu.make_async_copy(g_hbm.at[src_v.at[a + 1]],
                                  rows_v.at[1], sem1).wait()
            pltpu.async_copy(g_hbm.at[src_v.at[a + 2]], rows_v.at[0], sem0)
            pltpu.sync_copy(rows_v.at[1], acc_sh.at[dst_v.at[a + 1]], add=True)
            return carry

        lax.fori_loop(0, _KC2 // 2 - 1, step, 0)
        a = _KC2 - 2
        pltpu.make_async_copy(g_hbm.at[src_v.at[a]], rows_v.at[0], sem0).wait()
        pltpu.async_copy(g_hbm.at[src_v.at[a + 1]], rows_v.at[1], sem1)
        pltpu.sync_copy(rows_v.at[0], acc_sh.at[dst_v.at[a]], add=True)
        pltpu.make_async_copy(g_hbm.at[src_v.at[a + 1]],
                              rows_v.at[1], sem1).wait()
        pltpu.sync_copy(rows_v.at[1], acc_sh.at[dst_v.at[a + 1]], add=True)

    @pl.when(cid == 0)
    def _():
        run_half(g0_hbm)

    @pl.when(cid == 1)
    def _():
        run_half(g1_hbm)

    plsc.subcore_barrier()
    pltpu.sync_copy(acc_sh.at[pl.ds(r0, _RPT)], out_hbm.at[cid, pl.ds(r0, _RPT)])

    @pl.when(sid == 0)
    def _():
        pltpu.sync_copy(acc_sh.at[pl.ds(_NS * _RPT, _TAIL)],
                        out_hbm.at[cid, pl.ds(_NS * _RPT, _TAIL)])


def _mm_body(x_ref, w_ref, cnt_ref, g0_ref, g1_ref):
    h = jnp.dot(x_ref[...], w_ref[...],
                preferred_element_type=jnp.float32,
                precision=lax.Precision.HIGHEST)
    c = cnt_ref[...]
    dinv = lax.rsqrt(c[0] + c[1] + 1.0)
    g = h * dinv
    g0_ref[...] = g[:, :_DH]
    g1_ref[...] = g[:, _DH:]


_tc_g = pl.pallas_call(
    _mm_body,
    grid=(_GRID,),
    in_specs=[
        pl.BlockSpec((_BR, _D), lambda i: (i, 0)),
        pl.BlockSpec((_D, _D), lambda i: (0, 0)),
        pl.BlockSpec((_NC, _BR, 1), lambda i: (0, i, 0)),
    ],
    out_specs=[
        pl.BlockSpec((_BR, _DH), lambda i: (i, 0)),
        pl.BlockSpec((_BR, _DH), lambda i: (i, 0)),
    ],
    out_shape=[
        jax.ShapeDtypeStruct((_N, _DH), jnp.float32),
        jax.ShapeDtypeStruct((_N, _DH), jnp.float32),
    ],
)


def _ep_body(p_ref, g0_ref, g1_ref, cnt_ref, b_ref, o_ref):
    c = cnt_ref[...]
    dinv = lax.rsqrt(c[0] + c[1] + 1.0)
    s0 = (p_ref[0] + g0_ref[...]) * dinv + b_ref[:, :_DH]
    s1 = (p_ref[1] + g1_ref[...]) * dinv + b_ref[:, _DH:]
    u = jnp.concatenate([s0, s1], axis=1)
    o_ref[...] = u * 0.5 * (1.0 + lax.erf(u * (2.0 ** -0.5)))


_tc_ep = pl.pallas_call(
    _ep_body,
    grid=(_GRID,),
    in_specs=[
        pl.BlockSpec((_NC, _BR, _DH), lambda i: (0, i, 0)),
        pl.BlockSpec((_BR, _DH), lambda i: (i, 0)),
        pl.BlockSpec((_BR, _DH), lambda i: (i, 0)),
        pl.BlockSpec((_NC, _BR, 1), lambda i: (0, i, 0)),
        pl.BlockSpec((1, _D), lambda i: (0, 0)),
    ],
    out_specs=pl.BlockSpec((_BR, _D), lambda i: (i, 0)),
    out_shape=jax.ShapeDtypeStruct((_N, _D), jnp.float32),
)


def kernel(x, edge_index, W, b):
    src = edge_index[0].reshape(_NS, _KC2, _CB)
    dst = edge_index[1].reshape(_NS, _KC2, _CB)
    dst_deg = edge_index[1].reshape(_NW, _KC, _CB)
    zn = jnp.zeros((_N,), jnp.float32)
    z2 = jnp.zeros((_N, _DH), jnp.float32)
    cnt = _sc_degree(dst_deg, zn)
    cnt3 = cnt.reshape(_NC, _N, 1)
    g0, g1 = _tc_g(x, W, cnt3)
    p = _sc_scatter(g0, g1, src, dst, z2)
    return _tc_ep(p, g0, g1, cnt3, b.reshape(1, _D))


# speedup vs baseline: 33.4969x; 1.4015x over previous
import functools

import jax
import jax.numpy as jnp
from jax import lax
from jax.experimental import pallas as pl
from jax.experimental.pallas import tpu as pltpu
from jax.experimental.pallas import tpu_sc as plsc

_N = 10000
_E = 320000
_D = 128
_DH = _D // 2
_NC = 2
_NS = 16
_NW = _NC * _NS
_EPW = _E // _NW
_CB = 80
_KC = _EPW // _CB
_EPT = _E // _NS
_KC2 = _EPT // _CB
_RPT = 624
_TAIL = _N - _NS * _RPT
_BR = 1000
_GRID = _N // _BR

_mesh = plsc.VectorSubcoreMesh(
    core_axis_name="c", subcore_axis_name="s",
    num_cores=_NC, num_subcores=_NS,
)

_sc_params = pltpu.CompilerParams(use_tc_tiling_on_sc=False)


@functools.partial(
    pl.kernel,
    out_type=jax.ShapeDtypeStruct((_NC, _N), jnp.float32),
    mesh=_mesh,
    scratch_types=[
        pltpu.VMEM((_KC, _CB), jnp.int32),
        pltpu.VMEM((_CB,), jnp.float32),
        pltpu.VMEM_SHARED((_N,), jnp.float32),
        pltpu.SemaphoreType.DMA,
        pltpu.SemaphoreType.DMA,
        pltpu.SemaphoreType.DMA,
        pltpu.SemaphoreType.DMA,
    ],
    compiler_params=_sc_params,
)
def _sc_degree(dst_hbm, zn_hbm, out_hbm, dst_v, ones_v, cnt_sh,
               ds0, ds1, ds2, ds3):
    cid = lax.axis_index("c")
    sid = lax.axis_index("s")
    pltpu.sync_copy(dst_hbm.at[sid, pl.ds(cid * _KC, _KC)], dst_v)

    def fill(i, carry):
        ones_v[pl.ds(i * 16, 16)] = jnp.ones((16,), jnp.float32)
        return carry

    lax.fori_loop(0, _CB // 16, fill, 0)

    @pl.when(sid == 0)
    def _():
        pltpu.sync_copy(zn_hbm, cnt_sh)

    plsc.subcore_barrier()

    dsems = (ds0, ds1, ds2, ds3)

    def fire(c, b):
        pltpu.async_copy(ones_v, cnt_sh.at[dst_v.at[c]], dsems[b], add=True)

    def drain(c, b):
        pltpu.make_async_copy(ones_v, cnt_sh.at[dst_v.at[c]],
                              dsems[b]).wait()

    for b in range(4):
        fire(b, b)

    def grp(g, carry):
        for b in range(4):
            c = 4 * g + 4 + b
            drain(c - 4, b)
            fire(c, b)
        return carry

    lax.fori_loop(0, (_KC - 5) // 4, grp, 0)
    drain(_KC - 5, 0)
    fire(_KC - 1, 0)
    drain(_KC - 4, 1)
    drain(_KC - 3, 2)
    drain(_KC - 2, 3)
    drain(_KC - 1, 0)
    plsc.subcore_barrier()

    @pl.when(sid == 0)
    def _():
        pltpu.sync_copy(cnt_sh, out_hbm.at[cid])


@functools.partial(
    pl.kernel,
    out_type=jax.ShapeDtypeStruct((_NC, _N, _DH), jnp.float32),
    mesh=_mesh,
    scratch_types=[
        pltpu.VMEM((_KC2, _CB), jnp.int32),
        pltpu.VMEM((_KC2, _CB), jnp.int32),
        pltpu.VMEM((4, _CB, _DH), jnp.float32),
        pltpu.VMEM_SHARED((_N, _DH), jnp.float32),
        pltpu.SemaphoreType.DMA,
        pltpu.SemaphoreType.DMA,
        pltpu.SemaphoreType.DMA,
        pltpu.SemaphoreType.DMA,
        pltpu.SemaphoreType.DMA,
        pltpu.SemaphoreType.DMA,
        pltpu.SemaphoreType.DMA,
        pltpu.SemaphoreType.DMA,
    ],
    compiler_params=_sc_params,
)
def _sc_scatter(g0_hbm, g1_hbm, src_hbm, dst_hbm, z2_hbm, out_hbm,
                src_v, dst_v, rows_v, acc_sh,
                gs0, gs1, gs2, gs3, ss0, ss1, ss2, ss3):
    cid = lax.axis_index("c")
    sid = lax.axis_index("s")
    pltpu.sync_copy(src_hbm.at[sid], src_v)
    pltpu.sync_copy(dst_hbm.at[sid], dst_v)
    r0 = sid * _RPT
    pltpu.sync_copy(z2_hbm.at[pl.ds(r0, _RPT)], acc_sh.at[pl.ds(r0, _RPT)])

    @pl.when(sid == 0)
    def _():
        pltpu.sync_copy(z2_hbm.at[pl.ds(_NS * _RPT, _TAIL)],
                        acc_sh.at[pl.ds(_NS * _RPT, _TAIL)])

    plsc.subcore_barrier()

    gsems = (gs0, gs1, gs2, gs3)
    ssems = (ss0, ss1, ss2, ss3)

    def run_half(g_hbm):
        def gather(c, b):
            pltpu.async_copy(g_hbm.at[src_v.at[c]], rows_v.at[b], gsems[b])

        def wait_gather(c, b):
            pltpu.make_async_copy(g_hbm.at[src_v.at[c]],
                                  rows_v.at[b], gsems[b]).wait()

        def scatter(c, b):
            pltpu.async_copy(rows_v.at[b], acc_sh.at[dst_v.at[c]],
                             ssems[b], add=True)

        def wait_scatter(c, b):
            pltpu.make_async_copy(rows_v.at[b], acc_sh.at[dst_v.at[c]],
                                  ssems[b]).wait()

        gather(0, 0)
        gather(1, 1)

        def group(g, carry):
            for b in range(4):
                c = 4 * g + b
                wait_gather(c, b)
                scatter(c, b)
                b2 = (b + 2) % 4

                @pl.when(c >= 2)
                def _():
                    wait_scatter(c - 2, b2)

                gather(c + 2, b2)

            return carry

        lax.fori_loop(0, _KC2 // 4, group, 0)
        wait_gather(_KC2 - 2, 0)
        scatter(_KC2 - 2, 0)
        wait_gather(_KC2 - 1, 1)
        scatter(_KC2 - 1, 1)
        wait_scatter(_KC2 - 4, 2)
        wait_scatter(_KC2 - 3, 3)
        wait_scatter(_KC2 - 2, 0)
        wait_scatter(_KC2 - 1, 1)

    @pl.when(cid == 0)
    def _():
        run_half(g0_hbm)

    @pl.when(cid == 1)
    def _():
        run_half(g1_hbm)

    plsc.subcore_barrier()
    pltpu.sync_copy(acc_sh.at[pl.ds(r0, _RPT)], out_hbm.at[cid, pl.ds(r0, _RPT)])

    @pl.when(sid == 0)
    def _():
        pltpu.sync_copy(acc_sh.at[pl.ds(_NS * _RPT, _TAIL)],
                        out_hbm.at[cid, pl.ds(_NS * _RPT, _TAIL)])


def _mm_body(x_ref, w_ref, cnt_ref, g0_ref, g1_ref):
    h = jnp.dot(x_ref[...], w_ref[...],
                preferred_element_type=jnp.float32,
                precision=lax.Precision.HIGHEST)
    c = cnt_ref[...]
    dinv = lax.rsqrt(c[0] + c[1] + 1.0)
    g = h * dinv
    g0_ref[...] = g[:, :_DH]
    g1_ref[...] = g[:, _DH:]


_tc_g = pl.pallas_call(
    _mm_body,
    grid=(_GRID,),
    in_specs=[
        pl.BlockSpec((_BR, _D), lambda i: (i, 0)),
        pl.BlockSpec((_D, _D), lambda i: (0, 0)),
        pl.BlockSpec((_NC, _BR, 1), lambda i: (0, i, 0)),
    ],
    out_specs=[
        pl.BlockSpec((_BR, _DH), lambda i: (i, 0)),
        pl.BlockSpec((_BR, _DH), lambda i: (i, 0)),
    ],
    out_shape=[
        jax.ShapeDtypeStruct((_N, _DH), jnp.float32),
        jax.ShapeDtypeStruct((_N, _DH), jnp.float32),
    ],
)


def _ep_body(p_ref, g0_ref, g1_ref, cnt_ref, b_ref, o_ref):
    c = cnt_ref[...]
    dinv = lax.rsqrt(c[0] + c[1] + 1.0)
    s0 = (p_ref[0] + g0_ref[...]) * dinv + b_ref[:, :_DH]
    s1 = (p_ref[1] + g1_ref[...]) * dinv + b_ref[:, _DH:]
    u = jnp.concatenate([s0, s1], axis=1)
    o_ref[...] = u * 0.5 * (1.0 + lax.erf(u * (2.0 ** -0.5)))


_tc_ep = pl.pallas_call(
    _ep_body,
    grid=(_GRID,),
    in_specs=[
        pl.BlockSpec((_NC, _BR, _DH), lambda i: (0, i, 0)),
        pl.BlockSpec((_BR, _DH), lambda i: (i, 0)),
        pl.BlockSpec((_BR, _DH), lambda i: (i, 0)),
        pl.BlockSpec((_NC, _BR, 1), lambda i: (0, i, 0)),
        pl.BlockSpec((1, _D), lambda i: (0, 0)),
    ],
    out_specs=pl.BlockSpec((_BR, _D), lambda i: (i, 0)),
    out_shape=jax.ShapeDtypeStruct((_N, _D), jnp.float32),
)


def kernel(x, edge_index, W, b):
    src = edge_index[0].reshape(_NS, _KC2, _CB)
    dst = edge_index[1].reshape(_NS, _KC2, _CB)
    zn = jnp.zeros((_N,), jnp.float32)
    z2 = jnp.zeros((_N, _DH), jnp.float32)
    cnt = _sc_degree(dst, zn)
    cnt3 = cnt.reshape(_NC, _N, 1)
    g0, g1 = _tc_g(x, W, cnt3)
    p = _sc_scatter(g0, g1, src, dst, z2)
    return _tc_ep(p, g0, g1, cnt3, b.reshape(1, _D))


# speedup vs baseline: 41.5771x; 1.2412x over previous
import functools

import jax
import jax.numpy as jnp
from jax import lax
from jax.experimental import pallas as pl
from jax.experimental.pallas import tpu as pltpu
from jax.experimental.pallas import tpu_sc as plsc

_N = 10000
_E = 320000
_D = 128
_DH = _D // 2
_NC = 2
_NS = 16
_CB = 80
_EPT = _E // _NS
_KC2 = _EPT // _CB
_RPT = 624
_TAIL = _N - _NS * _RPT
_NH = _N // _NC
_DPT = 320
_DLAST = _NH - 15 * _DPT
_BR = 1000
_GRID = _N // _BR

_mesh = plsc.VectorSubcoreMesh(
    core_axis_name="c", subcore_axis_name="s",
    num_cores=_NC, num_subcores=_NS,
)

_sc_params = pltpu.CompilerParams(use_tc_tiling_on_sc=False)
_sc_params_nl = pltpu.CompilerParams(use_tc_tiling_on_sc=False,
                                     needs_layout_passes=False)


def _rsqrt_newton(x):
    xi = plsc.bitcast(x, jnp.int32)
    yi = jnp.int32(0x5F3759DF) - lax.shift_right_logical(xi, 1)
    y = plsc.bitcast(yi, jnp.float32)
    for _ in range(3):
        y = y * (1.5 - 0.5 * x * y * y)
    return y


@functools.partial(
    pl.kernel,
    out_type=jax.ShapeDtypeStruct((_N,), jnp.float32),
    mesh=_mesh,
    scratch_types=[
        pltpu.VMEM((_KC2, _CB), jnp.int32),
        pltpu.VMEM((_CB,), jnp.float32),
        pltpu.VMEM((_DPT,), jnp.float32),
        pltpu.VMEM_SHARED((_N,), jnp.float32),
        pltpu.SemaphoreType.DMA,
        pltpu.SemaphoreType.DMA,
        pltpu.SemaphoreType.DMA,
        pltpu.SemaphoreType.DMA,
    ],
    compiler_params=_sc_params_nl,
)
def _sc_degree(er_hbm, zn_hbm, out_hbm, dst_v, ones_v, dinv_v, cnt_sh,
               ds0, ds1, ds2, ds3):
    cid = lax.axis_index("c")
    sid = lax.axis_index("s")
    pltpu.sync_copy(er_hbm.at[1, sid], dst_v)

    def fill(i, carry):
        ones_v[pl.ds(i * 16, 16)] = jnp.ones((16,), jnp.float32)
        return carry

    lax.fori_loop(0, _CB // 16, fill, 0)

    @pl.when(sid == 0)
    def _():
        pltpu.sync_copy(zn_hbm, cnt_sh)

    plsc.subcore_barrier()

    dsems = (ds0, ds1, ds2, ds3)

    def fire(c, b):
        pltpu.async_copy(ones_v, cnt_sh.at[dst_v.at[c]], dsems[b], add=True)

    def drain(c, b):
        pltpu.make_async_copy(ones_v, cnt_sh.at[dst_v.at[c]],
                              dsems[b]).wait()

    for b in range(4):
        fire(b, b)

    def grp(g, carry):
        for b in range(4):
            c = 4 * g + 4 + b
            drain(c - 4, b)
            fire(c, b)
        return carry

    lax.fori_loop(0, (_KC2 - 4) // 4, grp, 0)
    drain(_KC2 - 6, 0)
    fire(_KC2 - 2, 0)
    drain(_KC2 - 5, 1)
    fire(_KC2 - 1, 1)
    drain(_KC2 - 4, 2)
    drain(_KC2 - 3, 3)
    drain(_KC2 - 2, 0)
    drain(_KC2 - 1, 1)
    plsc.subcore_barrier()

    base = cid * _NH + sid * _DPT

    def run(n_nodes, n_vec):
        pltpu.sync_copy(cnt_sh.at[pl.ds(base, n_nodes)],
                        dinv_v.at[pl.ds(0, n_nodes)])

        def body(i, carry):
            v = dinv_v[pl.ds(i * 16, 16)]
            dinv_v[pl.ds(i * 16, 16)] = _rsqrt_newton(v + 1.0)
            return carry

        lax.fori_loop(0, n_vec, body, 0)
        pltpu.sync_copy(dinv_v.at[pl.ds(0, n_nodes)],
                        out_hbm.at[pl.ds(base, n_nodes)])

    @pl.when(sid < _NS - 1)
    def _():
        run(_DPT, _DPT // 16)

    @pl.when(sid == _NS - 1)
    def _():
        run(_DLAST, (_DLAST + 15) // 16)


@functools.partial(
    pl.kernel,
    out_type=jax.ShapeDtypeStruct((_NC, _N, _DH), jnp.float32),
    mesh=_mesh,
    scratch_types=[
        pltpu.VMEM((_KC2, _CB), jnp.int32),
        pltpu.VMEM((_KC2, _CB), jnp.int32),
        pltpu.VMEM((8, _CB, _DH), jnp.float32),
        pltpu.VMEM_SHARED((_N, _DH), jnp.float32),
        [pltpu.SemaphoreType.DMA] * 8,
        [pltpu.SemaphoreType.DMA] * 8,
    ],
    compiler_params=_sc_params,
)
def _sc_scatter(g0_hbm, g1_hbm, er_hbm, z2_hbm, out_hbm,
                src_v, dst_v, rows_v, acc_sh, gsems, ssems):
    cid = lax.axis_index("c")
    sid = lax.axis_index("s")
    pltpu.sync_copy(er_hbm.at[0, sid], src_v)
    pltpu.sync_copy(er_hbm.at[1, sid], dst_v)
    r0 = sid * _RPT
    pltpu.sync_copy(z2_hbm.at[pl.ds(r0, _RPT)], acc_sh.at[pl.ds(r0, _RPT)])

    @pl.when(sid == 0)
    def _():
        pltpu.sync_copy(z2_hbm.at[pl.ds(_NS * _RPT, _TAIL)],
                        acc_sh.at[pl.ds(_NS * _RPT, _TAIL)])

    plsc.subcore_barrier()

    def run_half(g_hbm):
        def gather(c, b):
            pltpu.async_copy(g_hbm.at[src_v.at[c]], rows_v.at[b], gsems[b])

        def wait_gather(c, b):
            pltpu.make_async_copy(g_hbm.at[src_v.at[c]],
                                  rows_v.at[b], gsems[b]).wait()

        def scatter(c, b):
            pltpu.async_copy(rows_v.at[b], acc_sh.at[dst_v.at[c]],
                             ssems[b], add=True)

        def wait_scatter(c, b):
            pltpu.make_async_copy(rows_v.at[b], acc_sh.at[dst_v.at[c]],
                                  ssems[b]).wait()

        for c0 in range(4):
            gather(c0, c0)

        def group(g, carry):
            for b in range(8):
                c = 8 * g + b
                wait_gather(c, b)
                scatter(c, b)
                b4 = (b + 4) % 8

                @pl.when(c >= 4)
                def _():
                    wait_scatter(c - 4, b4)

                @pl.when(c + 4 < _KC2)
                def _():
                    gather(c + 4, b4)

            return carry

        lax.fori_loop(0, _KC2 // 8, group, 0)
        wait_gather(_KC2 - 2, 0)
        scatter(_KC2 - 2, 0)
        wait_gather(_KC2 - 1, 1)
        scatter(_KC2 - 1, 1)
        wait_scatter(_KC2 - 6, 4)
        wait_scatter(_KC2 - 5, 5)
        wait_scatter(_KC2 - 4, 6)
        wait_scatter(_KC2 - 3, 7)
        wait_scatter(_KC2 - 2, 0)
        wait_scatter(_KC2 - 1, 1)

    @pl.when(cid == 0)
    def _():
        run_half(g0_hbm)

    @pl.when(cid == 1)
    def _():
        run_half(g1_hbm)

    plsc.subcore_barrier()
    pltpu.sync_copy(acc_sh.at[pl.ds(r0, _RPT)], out_hbm.at[cid, pl.ds(r0, _RPT)])

    @pl.when(sid == 0)
    def _():
        pltpu.sync_copy(acc_sh.at[pl.ds(_NS * _RPT, _TAIL)],
                        out_hbm.at[cid, pl.ds(_NS * _RPT, _TAIL)])


def _mm_body(x_ref, w_ref, dv_ref, g0_ref, g1_ref):
    h = jnp.dot(x_ref[...], w_ref[...],
                preferred_element_type=jnp.float32,
                precision=lax.Precision.HIGHEST)
    g = h * dv_ref[...]
    g0_ref[...] = g[:, :_DH]
    g1_ref[...] = g[:, _DH:]


_tc_g = pl.pallas_call(
    _mm_body,
    grid=(_GRID,),
    in_specs=[
        pl.BlockSpec((_BR, _D), lambda i: (i, 0)),
        pl.BlockSpec((_D, _D), lambda i: (0, 0)),
        pl.BlockSpec((_BR, _D), lambda i: (i, 0)),
    ],
    out_specs=[
        pl.BlockSpec((_BR, _DH), lambda i: (i, 0)),
        pl.BlockSpec((_BR, _DH), lambda i: (i, 0)),
    ],
    out_shape=[
        jax.ShapeDtypeStruct((_N, _DH), jnp.float32),
        jax.ShapeDtypeStruct((_N, _DH), jnp.float32),
    ],
)


def _ep_body(p_ref, g0_ref, g1_ref, dv_ref, b_ref, o_ref):
    s0 = p_ref[0] + g0_ref[...]
    s1 = p_ref[1] + g1_ref[...]
    u = jnp.concatenate([s0, s1], axis=1) * dv_ref[...] + b_ref[...]
    o_ref[...] = u * 0.5 * (1.0 + lax.erf(u * (2.0 ** -0.5)))


_tc_ep = pl.pallas_call(
    _ep_body,
    grid=(_GRID,),
    in_specs=[
        pl.BlockSpec((_NC, _BR, _DH), lambda i: (0, i, 0)),
        pl.BlockSpec((_BR, _DH), lambda i: (i, 0)),
        pl.BlockSpec((_BR, _DH), lambda i: (i, 0)),
        pl.BlockSpec((_BR, _D), lambda i: (i, 0)),
        pl.BlockSpec((1, _D), lambda i: (0, 0)),
    ],
    out_specs=pl.BlockSpec((_BR, _D), lambda i: (i, 0)),
    out_shape=jax.ShapeDtypeStruct((_N, _D), jnp.float32),
)


def kernel(x, edge_index, W, b):
    er = edge_index.reshape(2, _NS, _KC2, _CB)
    zn = jnp.zeros((_N,), jnp.float32)
    z2 = jnp.zeros((_N, _DH), jnp.float32)
    dinv = _sc_degree(er, zn)
    dinvb = jnp.broadcast_to(dinv[:, None], (_N, _D))
    g0, g1 = _tc_g(x, W, dinvb)
    p = _sc_scatter(g0, g1, er, z2)
    return _tc_ep(p, g0, g1, dinvb, b.reshape(1, _D))


# speedup vs baseline: 42.5264x; 1.0228x over previous
import functools

import jax
import jax.numpy as jnp
from jax import lax
from jax.experimental import pallas as pl
from jax.experimental.pallas import tpu as pltpu
from jax.experimental.pallas import tpu_sc as plsc

_N = 10000
_E = 320000
_D = 128
_DH = _D // 2
_NC = 2
_NS = 16
_CB = 96
_KC2 = 216
_EPT = _KC2 * _CB
_SLPT = _N // _NS
_DMPT = _EPT - _E // _NS - _SLPT
_NA = _N + 16
_RPT = 624
_ATAIL = _NA - _NS * _RPT
_OTAIL = _N - _NS * _RPT
_NH = _N // _NC
_DPT = 320
_DLAST = _NH - 15 * _DPT
_BR = 2000
_GRID = _N // _BR

_mesh = plsc.VectorSubcoreMesh(
    core_axis_name="c", subcore_axis_name="s",
    num_cores=_NC, num_subcores=_NS,
)

_sc_params = pltpu.CompilerParams(use_tc_tiling_on_sc=False)
_sc_params_nl = pltpu.CompilerParams(use_tc_tiling_on_sc=False,
                                     needs_layout_passes=False)


def _rsqrt_newton(x):
    xi = plsc.bitcast(x, jnp.int32)
    yi = jnp.int32(0x5F3759DF) - lax.shift_right_logical(xi, 1)
    y = plsc.bitcast(yi, jnp.float32)
    for _ in range(3):
        y = y * (1.5 - 0.5 * x * y * y)
    return y


@functools.partial(
    pl.kernel,
    out_type=jax.ShapeDtypeStruct((_N,), jnp.float32),
    mesh=_mesh,
    scratch_types=[
        pltpu.VMEM((_KC2, _CB), jnp.int32),
        pltpu.VMEM((_CB,), jnp.float32),
        pltpu.VMEM((_DPT,), jnp.float32),
        pltpu.VMEM_SHARED((_NA,), jnp.float32),
        pltpu.SemaphoreType.DMA,
        pltpu.SemaphoreType.DMA,
        pltpu.SemaphoreType.DMA,
        pltpu.SemaphoreType.DMA,
    ],
    compiler_params=_sc_params_nl,
)
def _sc_degree(er_hbm, zn_hbm, out_hbm, dst_v, ones_v, dinv_v, cnt_sh,
               ds0, ds1, ds2, ds3):
    cid = lax.axis_index("c")
    sid = lax.axis_index("s")
    pltpu.sync_copy(er_hbm.at[1, sid], dst_v)

    def fill(i, carry):
        ones_v[pl.ds(i * 16, 16)] = jnp.ones((16,), jnp.float32)
        return carry

    lax.fori_loop(0, _CB // 16, fill, 0)

    @pl.when(sid == 0)
    def _():
        pltpu.sync_copy(zn_hbm, cnt_sh)

    plsc.subcore_barrier()

    dsems = (ds0, ds1, ds2, ds3)

    def fire(c, b):
        pltpu.async_copy(ones_v, cnt_sh.at[dst_v.at[c]], dsems[b], add=True)

    def drain(c, b):
        pltpu.make_async_copy(ones_v, cnt_sh.at[dst_v.at[c]],
                              dsems[b]).wait()

    for b in range(4):
        fire(b, b)

    def grp(g, carry):
        for b in range(4):
            c = 4 * g + 4 + b
            drain(c - 4, b)
            fire(c, b)
        return carry

    lax.fori_loop(0, (_KC2 - 4) // 4, grp, 0)
    drain(_KC2 - 4, 0)
    drain(_KC2 - 3, 1)
    drain(_KC2 - 2, 2)
    drain(_KC2 - 1, 3)
    plsc.subcore_barrier()

    base = cid * _NH + sid * _DPT

    def run(n_nodes, n_vec):
        pltpu.sync_copy(cnt_sh.at[pl.ds(base, n_nodes)],
                        dinv_v.at[pl.ds(0, n_nodes)])

        def body(i, carry):
            v = dinv_v[pl.ds(i * 16, 16)]
            dinv_v[pl.ds(i * 16, 16)] = _rsqrt_newton(v)
            return carry

        lax.fori_loop(0, n_vec, body, 0)
        pltpu.sync_copy(dinv_v.at[pl.ds(0, n_nodes)],
                        out_hbm.at[pl.ds(base, n_nodes)])

    @pl.when(sid < _NS - 1)
    def _():
        run(_DPT, _DPT // 16)

    @pl.when(sid == _NS - 1)
    def _():
        run(_DLAST, (_DLAST + 15) // 16)


@functools.partial(
    pl.kernel,
    out_type=jax.ShapeDtypeStruct((_NC, _N, _DH), jnp.float32),
    mesh=_mesh,
    scratch_types=[
        pltpu.VMEM((_KC2, _CB), jnp.int32),
        pltpu.VMEM((_KC2, _CB), jnp.int32),
        pltpu.VMEM((8, _CB, _DH), jnp.float32),
        pltpu.VMEM_SHARED((_NA, _DH), jnp.float32),
        [pltpu.SemaphoreType.DMA] * 8,
        [pltpu.SemaphoreType.DMA] * 8,
    ],
    compiler_params=_sc_params,
)
def _sc_scatter(g0_hbm, g1_hbm, er_hbm, z2_hbm, out_hbm,
                src_v, dst_v, rows_v, acc_sh, gsems, ssems):
    cid = lax.axis_index("c")
    sid = lax.axis_index("s")
    pltpu.sync_copy(er_hbm.at[0, sid], src_v)
    pltpu.sync_copy(er_hbm.at[1, sid], dst_v)
    r0 = sid * _RPT
    pltpu.sync_copy(z2_hbm.at[pl.ds(r0, _RPT)], acc_sh.at[pl.ds(r0, _RPT)])

    @pl.when(sid == 0)
    def _():
        pltpu.sync_copy(z2_hbm.at[pl.ds(_NS * _RPT, _ATAIL)],
                        acc_sh.at[pl.ds(_NS * _RPT, _ATAIL)])

    plsc.subcore_barrier()

    def run_half(g_hbm):
        def gather(c, b):
            pltpu.async_copy(g_hbm.at[src_v.at[c]], rows_v.at[b], gsems[b])

        def wait_gather(c, b):
            pltpu.make_async_copy(g_hbm.at[src_v.at[c]],
                                  rows_v.at[b], gsems[b]).wait()

        def scatter(c, b):
            pltpu.async_copy(rows_v.at[b], acc_sh.at[dst_v.at[c]],
                             ssems[b], add=True)

        def wait_scatter(c, b):
            pltpu.make_async_copy(rows_v.at[b], acc_sh.at[dst_v.at[c]],
                                  ssems[b]).wait()

        for c0 in range(4):
            gather(c0, c0)

        def group(g, carry):
            for b in range(8):
                c = 8 * g + b
                wait_gather(c, b)
                scatter(c, b)
                b4 = (b + 4) % 8

                @pl.when(c >= 4)
                def _():
                    wait_scatter(c - 4, b4)

                @pl.when(c + 4 < _KC2)
                def _():
                    gather(c + 4, b4)

            return carry

        lax.fori_loop(0, _KC2 // 8, group, 0)
        wait_scatter(_KC2 - 4, 4)
        wait_scatter(_KC2 - 3, 5)
        wait_scatter(_KC2 - 2, 6)
        wait_scatter(_KC2 - 1, 7)

    @pl.when(cid == 0)
    def _():
        run_half(g0_hbm)

    @pl.when(cid == 1)
    def _():
        run_half(g1_hbm)

    plsc.subcore_barrier()
    pltpu.sync_copy(acc_sh.at[pl.ds(r0, _RPT)], out_hbm.at[cid, pl.ds(r0, _RPT)])

    @pl.when(sid == 0)
    def _():
        pltpu.sync_copy(acc_sh.at[pl.ds(_NS * _RPT, _OTAIL)],
                        out_hbm.at[cid, pl.ds(_NS * _RPT, _OTAIL)])


def _mm_body(x_ref, w_ref, dv_ref, g0_ref, g1_ref):
    h = jnp.dot(x_ref[...], w_ref[...],
                preferred_element_type=jnp.float32,
                precision=lax.Precision.HIGHEST)
    g = h * dv_ref[...]
    g0_ref[...] = g[:, :_DH]
    g1_ref[...] = g[:, _DH:]


_tc_g = pl.pallas_call(
    _mm_body,
    grid=(_GRID,),
    in_specs=[
        pl.BlockSpec((_BR, _D), lambda i: (i, 0)),
        pl.BlockSpec((_D, _D), lambda i: (0, 0)),
        pl.BlockSpec((_BR, _D), lambda i: (i, 0)),
    ],
    out_specs=[
        pl.BlockSpec((_BR, _DH), lambda i: (i, 0)),
        pl.BlockSpec((_BR, _DH), lambda i: (i, 0)),
    ],
    out_shape=[
        jax.ShapeDtypeStruct((_N, _DH), jnp.float32),
        jax.ShapeDtypeStruct((_N, _DH), jnp.float32),
    ],
)


def _ep_body(p_ref, dv_ref, b_ref, o_ref):
    u = jnp.concatenate([p_ref[0], p_ref[1]], axis=1) * dv_ref[...] + b_ref[...]
    o_ref[...] = u * 0.5 * (1.0 + lax.erf(u * (2.0 ** -0.5)))


_tc_ep = pl.pallas_call(
    _ep_body,
    grid=(_GRID,),
    in_specs=[
        pl.BlockSpec((_NC, _BR, _DH), lambda i: (0, i, 0)),
        pl.BlockSpec((_BR, _D), lambda i: (i, 0)),
        pl.BlockSpec((1, _D), lambda i: (0, 0)),
    ],
    out_specs=pl.BlockSpec((_BR, _D), lambda i: (i, 0)),
    out_shape=jax.ShapeDtypeStruct((_N, _D), jnp.float32),
)


def kernel(x, edge_index, W, b):
    e2 = edge_index.reshape(2, _NS, _E // _NS)
    loop = jnp.arange(_N, dtype=jnp.int32).reshape(_NS, _SLPT)
    sl = jnp.stack([loop, loop])
    dmi = jnp.arange(_NS * _DMPT, dtype=jnp.int32).reshape(_NS, _DMPT)
    dm = jnp.stack([dmi % _N, _N + (dmi % 16)])
    er = jnp.concatenate([e2, sl, dm], axis=2).reshape(2, _NS, _KC2, _CB)
    zn = jnp.zeros((_NA,), jnp.float32)
    z2 = jnp.zeros((_NA, _DH), jnp.float32)
    dinv = _sc_degree(er, zn)
    dinvb = jnp.broadcast_to(dinv[:, None], (_N, _D))
    g0, g1 = _tc_g(x, W, dinvb)
    p = _sc_scatter(g0, g1, er, z2)
    return _tc_ep(p, dinvb, b.reshape(1, _D))


# speedup vs baseline: 47.2990x; 1.1122x over previous
import functools

import jax
import jax.numpy as jnp
from jax import lax
from jax.experimental import pallas as pl
from jax.experimental.pallas import tpu as pltpu
from jax.experimental.pallas import tpu_sc as plsc

_N = 10000
_E = 320000
_D = 128
_DH = _D // 2
_NC = 2
_NS = 16
_CB = 96
_KC2 = 216
_EPT = _KC2 * _CB
_SLPT = _N // _NS
_DMPT = _EPT - _E // _NS - _SLPT
_NA = _N + 16
_RPT = 624
_ATAIL = _NA - _NS * _RPT
_OTAIL = _N - _NS * _RPT
_NH = _N // _NC
_DPT = 320
_DLAST = _NH - 15 * _DPT
_BR = 2000
_GRID = _N // _BR

_mesh = plsc.VectorSubcoreMesh(
    core_axis_name="c", subcore_axis_name="s",
    num_cores=_NC, num_subcores=_NS,
)

_sc_params = pltpu.CompilerParams(use_tc_tiling_on_sc=False)
_sc_params_nl = pltpu.CompilerParams(use_tc_tiling_on_sc=False,
                                     needs_layout_passes=False)


def _rsqrt_newton(x):
    xi = plsc.bitcast(x, jnp.int32)
    yi = jnp.int32(0x5F3759DF) - lax.shift_right_logical(xi, 1)
    y = plsc.bitcast(yi, jnp.float32)
    for _ in range(3):
        y = y * (1.5 - 0.5 * x * y * y)
    return y


@functools.partial(
    pl.kernel,
    out_type=jax.ShapeDtypeStruct((_N,), jnp.float32),
    mesh=_mesh,
    scratch_types=[
        pltpu.VMEM((_KC2, _CB), jnp.int32),
        pltpu.VMEM((_CB,), jnp.float32),
        pltpu.VMEM((_DPT,), jnp.float32),
        pltpu.VMEM_SHARED((_NA,), jnp.float32),
        pltpu.SemaphoreType.DMA,
        pltpu.SemaphoreType.DMA,
        pltpu.SemaphoreType.DMA,
        pltpu.SemaphoreType.DMA,
    ],
    compiler_params=_sc_params_nl,
)
def _sc_degree(er_hbm, zn_hbm, out_hbm, dst_v, ones_v, dinv_v, cnt_sh,
               ds0, ds1, ds2, ds3):
    cid = lax.axis_index("c")
    sid = lax.axis_index("s")
    pltpu.sync_copy(er_hbm.at[1, sid], dst_v)

    def fill(i, carry):
        ones_v[pl.ds(i * 16, 16)] = jnp.ones((16,), jnp.float32)
        return carry

    lax.fori_loop(0, _CB // 16, fill, 0)

    @pl.when(sid == 0)
    def _():
        pltpu.sync_copy(zn_hbm, cnt_sh)

    plsc.subcore_barrier()

    dsems = (ds0, ds1, ds2, ds3)

    def fire(c, b):
        pltpu.async_copy(ones_v, cnt_sh.at[dst_v.at[c]], dsems[b], add=True)

    def drain(c, b):
        pltpu.make_async_copy(ones_v, cnt_sh.at[dst_v.at[c]],
                              dsems[b]).wait()

    for b in range(4):
        fire(b, b)

    def grp(g, carry):
        for b in range(4):
            c = 4 * g + 4 + b
            drain(c - 4, b)
            fire(c, b)
        return carry

    lax.fori_loop(0, (_KC2 - 4) // 4, grp, 0)
    drain(_KC2 - 4, 0)
    drain(_KC2 - 3, 1)
    drain(_KC2 - 2, 2)
    drain(_KC2 - 1, 3)
    plsc.subcore_barrier()

    base = cid * _NH + sid * _DPT

    def run(n_nodes, n_vec):
        pltpu.sync_copy(cnt_sh.at[pl.ds(base, n_nodes)],
                        dinv_v.at[pl.ds(0, n_nodes)])

        def body(i, carry):
            v = dinv_v[pl.ds(i * 16, 16)]
            dinv_v[pl.ds(i * 16, 16)] = _rsqrt_newton(v)
            return carry

        lax.fori_loop(0, n_vec, body, 0)
        pltpu.sync_copy(dinv_v.at[pl.ds(0, n_nodes)],
                        out_hbm.at[pl.ds(base, n_nodes)])

    @pl.when(sid < _NS - 1)
    def _():
        run(_DPT, _DPT // 16)

    @pl.when(sid == _NS - 1)
    def _():
        run(_DLAST, (_DLAST + 15) // 16)


@functools.partial(
    pl.kernel,
    out_type=jax.ShapeDtypeStruct((_N, _D), jnp.float32),
    mesh=_mesh,
    scratch_types=[
        pltpu.VMEM((_KC2, _CB), jnp.int32),
        pltpu.VMEM((_KC2, _CB), jnp.int32),
        pltpu.VMEM((8, _CB, _DH), jnp.float32),
        pltpu.VMEM_SHARED((_NA, _DH), jnp.float32),
        [pltpu.SemaphoreType.DMA] * 8,
        [pltpu.SemaphoreType.DMA] * 8,
    ],
    compiler_params=_sc_params,
)
def _sc_scatter(g_hbm, er_hbm, z2_hbm, out_hbm,
                src_v, dst_v, rows_v, acc_sh, gsems, ssems):
    cid = lax.axis_index("c")
    sid = lax.axis_index("s")
    pltpu.sync_copy(er_hbm.at[0, sid], src_v)
    pltpu.sync_copy(er_hbm.at[1, sid], dst_v)

    def xrow(r, carry):
        def xvec(j, carry2):
            v = src_v[r, pl.ds(j * 16, 16)]
            src_v[r, pl.ds(j * 16, 16)] = v * 2 + cid
            return carry2

        return lax.fori_loop(0, _CB // 16, xvec, carry)

    lax.fori_loop(0, _KC2, xrow, 0)
    r0 = sid * _RPT
    pltpu.sync_copy(z2_hbm.at[pl.ds(r0, _RPT)], acc_sh.at[pl.ds(r0, _RPT)])

    @pl.when(sid == 0)
    def _():
        pltpu.sync_copy(z2_hbm.at[pl.ds(_NS * _RPT, _ATAIL)],
                        acc_sh.at[pl.ds(_NS * _RPT, _ATAIL)])

    plsc.subcore_barrier()

    def gather(c, b):
        pltpu.async_copy(g_hbm.at[src_v.at[c]], rows_v.at[b], gsems[b])

    def wait_gather(c, b):
        pltpu.make_async_copy(g_hbm.at[src_v.at[c]],
                              rows_v.at[b], gsems[b]).wait()

    def scatter(c, b):
        pltpu.async_copy(rows_v.at[b], acc_sh.at[dst_v.at[c]],
                         ssems[b], add=True)

    def wait_scatter(c, b):
        pltpu.make_async_copy(rows_v.at[b], acc_sh.at[dst_v.at[c]],
                              ssems[b]).wait()

    for c0 in range(4):
        gather(c0, c0)

    def group(g, carry):
        for b in range(8):
            c = 8 * g + b
            wait_gather(c, b)
            scatter(c, b)
            b4 = (b + 4) % 8

            @pl.when(c >= 4)
            def _():
                wait_scatter(c - 4, b4)

            @pl.when(c + 4 < _KC2)
            def _():
                gather(c + 4, b4)

        return carry

    lax.fori_loop(0, _KC2 // 8, group, 0)
    wait_scatter(_KC2 - 4, 4)
    wait_scatter(_KC2 - 3, 5)
    wait_scatter(_KC2 - 2, 6)
    wait_scatter(_KC2 - 1, 7)

    plsc.subcore_barrier()
    pltpu.sync_copy(acc_sh.at[pl.ds(r0, _RPT)],
                    out_hbm.at[pl.ds(r0, _RPT), pl.ds(cid * _DH, _DH)])

    @pl.when(sid == 0)
    def _():
        pltpu.sync_copy(acc_sh.at[pl.ds(_NS * _RPT, _OTAIL)],
                        out_hbm.at[pl.ds(_NS * _RPT, _OTAIL),
                                   pl.ds(cid * _DH, _DH)])


def _mm_body(x_ref, w_ref, dv_ref, g_ref):
    h = jnp.dot(x_ref[...], w_ref[...],
                preferred_element_type=jnp.float32,
                precision=lax.Precision.HIGHEST)
    g_ref[...] = h * dv_ref[...]


_tc_g = pl.pallas_call(
    _mm_body,
    grid=(_GRID,),
    in_specs=[
        pl.BlockSpec((_BR, _D), lambda i: (i, 0)),
        pl.BlockSpec((_D, _D), lambda i: (0, 0)),
        pl.BlockSpec((_BR, _D), lambda i: (i, 0)),
    ],
    out_specs=pl.BlockSpec((_BR, _D), lambda i: (i, 0)),
    out_shape=jax.ShapeDtypeStruct((_N, _D), jnp.float32),
)


def _ep_body(p_ref, dv_ref, b_ref, o_ref):
    u = p_ref[...] * dv_ref[...] + b_ref[...]
    o_ref[...] = u * 0.5 * (1.0 + lax.erf(u * (2.0 ** -0.5)))


_tc_ep = pl.pallas_call(
    _ep_body,
    grid=(_GRID,),
    in_specs=[
        pl.BlockSpec((_BR, _D), lambda i: (i, 0)),
        pl.BlockSpec((_BR, _D), lambda i: (i, 0)),
        pl.BlockSpec((1, _D), lambda i: (0, 0)),
    ],
    out_specs=pl.BlockSpec((_BR, _D), lambda i: (i, 0)),
    out_shape=jax.ShapeDtypeStruct((_N, _D), jnp.float32),
)


def kernel(x, edge_index, W, b):
    e2 = edge_index.reshape(2, _NS, _E // _NS)
    loop = jnp.arange(_N, dtype=jnp.int32).reshape(_NS, _SLPT)
    sl = jnp.stack([loop, loop])
    dmi = jnp.arange(_NS * _DMPT, dtype=jnp.int32).reshape(_NS, _DMPT)
    dm = jnp.stack([dmi % _N, _N + (dmi % 16)])
    er = jnp.concatenate([e2, sl, dm], axis=2).reshape(2, _NS, _KC2, _CB)
    zn = jnp.zeros((_NA,), jnp.float32)
    z2 = jnp.zeros((_NA, _DH), jnp.float32)
    dinv = _sc_degree(er, zn)
    dinvb = jnp.broadcast_to(dinv[:, None], (_N, _D))
    g = _tc_g(x, W, dinvb)
    gview = g.reshape(2 * _N, _DH)
    p = _sc_scatter(gview, er, z2)
    return _tc_ep(p, dinvb, b.reshape(1, _D))


# speedup vs baseline: 47.8073x; 1.0107x over previous
import functools

import jax
import jax.numpy as jnp
from jax import lax
from jax.experimental import pallas as pl
from jax.experimental.pallas import tpu as pltpu
from jax.experimental.pallas import tpu_sc as plsc

_N = 10000
_E = 320000
_D = 128
_DH = _D // 2
_NC = 2
_NS = 16
_CB = 96
_KC2 = 216
_EPT = _KC2 * _CB
_SLPT = _N // _NS
_DMPT = _EPT - _E // _NS - _SLPT
_NA = _N + 16
_RPT = 624
_ATAIL = _NA - _NS * _RPT
_OTAIL = _N - _NS * _RPT
_NH = _N // _NC
_DPT = 320
_DLAST = _NH - 15 * _DPT
_BR = 2000
_GRID = _N // _BR

_mesh = plsc.VectorSubcoreMesh(
    core_axis_name="c", subcore_axis_name="s",
    num_cores=_NC, num_subcores=_NS,
)

_sc_params = pltpu.CompilerParams(use_tc_tiling_on_sc=False)
_sc_params_nl = pltpu.CompilerParams(use_tc_tiling_on_sc=False,
                                     needs_layout_passes=False)


def _rsqrt_newton(x):
    xi = plsc.bitcast(x, jnp.int32)
    yi = jnp.int32(0x5F3759DF) - lax.shift_right_logical(xi, 1)
    y = plsc.bitcast(yi, jnp.float32)
    for _ in range(3):
        y = y * (1.5 - 0.5 * x * y * y)
    return y


@functools.partial(
    pl.kernel,
    out_type=jax.ShapeDtypeStruct((_N,), jnp.float32),
    mesh=_mesh,
    scratch_types=[
        pltpu.VMEM((_KC2, _CB), jnp.int32),
        pltpu.VMEM((_CB,), jnp.float32),
        pltpu.VMEM((_DPT,), jnp.float32),
        pltpu.VMEM_SHARED((_NA,), jnp.float32),
        [pltpu.SemaphoreType.DMA] * 8,
    ],
    compiler_params=_sc_params_nl,
)
def _sc_degree(er_hbm, zn_hbm, out_hbm, dst_v, ones_v, dinv_v, cnt_sh,
               dsems):
    cid = lax.axis_index("c")
    sid = lax.axis_index("s")
    pltpu.sync_copy(er_hbm.at[1, sid], dst_v)

    def fill(i, carry):
        ones_v[pl.ds(i * 16, 16)] = jnp.ones((16,), jnp.float32)
        return carry

    lax.fori_loop(0, _CB // 16, fill, 0)

    @pl.when(sid == 0)
    def _():
        pltpu.sync_copy(zn_hbm, cnt_sh)

    plsc.subcore_barrier()

    def fire(c, b):
        pltpu.async_copy(ones_v, cnt_sh.at[dst_v.at[c]], dsems[b], add=True)

    def drain(c, b):
        pltpu.make_async_copy(ones_v, cnt_sh.at[dst_v.at[c]],
                              dsems[b]).wait()

    for b in range(8):
        fire(b, b)

    def grp(g, carry):
        for b in range(8):
            c = 8 * g + 8 + b
            drain(c - 8, b)
            fire(c, b)
        return carry

    lax.fori_loop(0, (_KC2 - 8) // 8, grp, 0)
    for b in range(8):
        drain(_KC2 - 8 + b, b)
    plsc.subcore_barrier()

    base = cid * _NH + sid * _DPT

    def run(n_nodes, n_vec):
        pltpu.sync_copy(cnt_sh.at[pl.ds(base, n_nodes)],
                        dinv_v.at[pl.ds(0, n_nodes)])

        def body(i, carry):
            v = dinv_v[pl.ds(i * 16, 16)]
            dinv_v[pl.ds(i * 16, 16)] = _rsqrt_newton(v)
            return carry

        lax.fori_loop(0, n_vec, body, 0)
        pltpu.sync_copy(dinv_v.at[pl.ds(0, n_nodes)],
                        out_hbm.at[pl.ds(base, n_nodes)])

    @pl.when(sid < _NS - 1)
    def _():
        run(_DPT, _DPT // 16)

    @pl.when(sid == _NS - 1)
    def _():
        run(_DLAST, (_DLAST + 15) // 16)


@functools.partial(
    pl.kernel,
    out_type=jax.ShapeDtypeStruct((_N, _D), jnp.float32),
    mesh=_mesh,
    scratch_types=[
        pltpu.VMEM((_KC2, _CB), jnp.int32),
        pltpu.VMEM((_KC2, _CB), jnp.int32),
        pltpu.VMEM((8, _CB, _DH), jnp.float32),
        pltpu.VMEM_SHARED((_NA, _DH), jnp.float32),
        [pltpu.SemaphoreType.DMA] * 8,
        [pltpu.SemaphoreType.DMA] * 8,
    ],
    compiler_params=_sc_params,
)
def _sc_scatter(g_hbm, er_hbm, z2_hbm, out_hbm,
                src_v, dst_v, rows_v, acc_sh, gsems, ssems):
    cid = lax.axis_index("c")
    sid = lax.axis_index("s")
    pltpu.sync_copy(er_hbm.at[0, sid], src_v)
    pltpu.sync_copy(er_hbm.at[1, sid], dst_v)

    def xrow(r, carry):
        def xvec(j, carry2):
            v = src_v[r, pl.ds(j * 16, 16)]
            src_v[r, pl.ds(j * 16, 16)] = v * 2 + cid
            return carry2

        return lax.fori_loop(0, _CB // 16, xvec, carry)

    lax.fori_loop(0, _KC2, xrow, 0)
    r0 = sid * _RPT
    pltpu.sync_copy(z2_hbm.at[pl.ds(r0, _RPT)], acc_sh.at[pl.ds(r0, _RPT)])

    @pl.when(sid == 0)
    def _():
        pltpu.sync_copy(z2_hbm.at[pl.ds(_NS * _RPT, _ATAIL)],
                        acc_sh.at[pl.ds(_NS * _RPT, _ATAIL)])

    plsc.subcore_barrier()

    def gather(c, b):
        pltpu.async_copy(g_hbm.at[src_v.at[c]], rows_v.at[b], gsems[b])

    def wait_gather(c, b):
        pltpu.make_async_copy(g_hbm.at[src_v.at[c]],
                              rows_v.at[b], gsems[b]).wait()

    def scatter(c, b):
        pltpu.async_copy(rows_v.at[b], acc_sh.at[dst_v.at[c]],
                         ssems[b], add=True)

    def wait_scatter(c, b):
        pltpu.make_async_copy(rows_v.at[b], acc_sh.at[dst_v.at[c]],
                              ssems[b]).wait()

    for c0 in range(4):
        gather(c0, c0)

    def group(g, carry):
        for b in range(8):
            c = 8 * g + b
            wait_gather(c, b)
            scatter(c, b)
            b4 = (b + 4) % 8

            @pl.when(c >= 4)
            def _():
                wait_scatter(c - 4, b4)

            @pl.when(c + 4 < _KC2)
            def _():
                gather(c + 4, b4)

        return carry

    lax.fori_loop(0, _KC2 // 8, group, 0)
    wait_scatter(_KC2 - 4, 4)
    wait_scatter(_KC2 - 3, 5)
    wait_scatter(_KC2 - 2, 6)
    wait_scatter(_KC2 - 1, 7)

    plsc.subcore_barrier()
    pltpu.sync_copy(acc_sh.at[pl.ds(r0, _RPT)],
                    out_hbm.at[pl.ds(r0, _RPT), pl.ds(cid * _DH, _DH)])

    @pl.when(sid == 0)
    def _():
        pltpu.sync_copy(acc_sh.at[pl.ds(_NS * _RPT, _OTAIL)],
                        out_hbm.at[pl.ds(_NS * _RPT, _OTAIL),
                                   pl.ds(cid * _DH, _DH)])


def _mm_body(x_ref, w_ref, h_ref):
    h_ref[...] = jnp.dot(x_ref[...], w_ref[...],
                         preferred_element_type=jnp.float32,
                         precision=lax.Precision.HIGHEST)


_tc_mm = pl.pallas_call(
    _mm_body,
    grid=(_GRID,),
    in_specs=[
        pl.BlockSpec((_BR, _D), lambda i: (i, 0)),
        pl.BlockSpec((_D, _D), lambda i: (0, 0)),
    ],
    out_specs=pl.BlockSpec((_BR, _D), lambda i: (i, 0)),
    out_shape=jax.ShapeDtypeStruct((_N, _D), jnp.float32),
)


def _scale_body(h_ref, dv_ref, g_ref):
    g_ref[...] = h_ref[...] * dv_ref[...]


_tc_scale = pl.pallas_call(
    _scale_body,
    grid=(_GRID,),
    in_specs=[
        pl.BlockSpec((_BR, _D), lambda i: (i, 0)),
        pl.BlockSpec((_BR, _D), lambda i: (i, 0)),
    ],
    out_specs=pl.BlockSpec((_BR, _D), lambda i: (i, 0)),
    out_shape=jax.ShapeDtypeStruct((_N, _D), jnp.float32),
)


def _ep_body(p_ref, dv_ref, b_ref, o_ref):
    u = p_ref[...] * dv_ref[...] + b_ref[...]
    o_ref[...] = u * 0.5 * (1.0 + lax.erf(u * (2.0 ** -0.5)))


_tc_ep = pl.pallas_call(
    _ep_body,
    grid=(_GRID,),
    in_specs=[
        pl.BlockSpec((_BR, _D), lambda i: (i, 0)),
        pl.BlockSpec((_BR, _D), lambda i: (i, 0)),
        pl.BlockSpec((1, _D), lambda i: (0, 0)),
    ],
    out_specs=pl.BlockSpec((_BR, _D), lambda i: (i, 0)),
    out_shape=jax.ShapeDtypeStruct((_N, _D), jnp.float32),
)


def kernel(x, edge_index, W, b):
    e2 = edge_index.reshape(2, _NS, _E // _NS)
    loop = jnp.arange(_N, dtype=jnp.int32).reshape(_NS, _SLPT)
    sl = jnp.stack([loop, loop])
    dmi = jnp.arange(_NS * _DMPT, dtype=jnp.int32).reshape(_NS, _DMPT)
    dm = jnp.stack([dmi % _N, _N + (dmi % 16)])
    er = jnp.concatenate([e2, sl, dm], axis=2).reshape(2, _NS, _KC2, _CB)
    zn = jnp.zeros((_NA,), jnp.float32)
    z2 = jnp.zeros((_NA, _DH), jnp.float32)
    h = _tc_mm(x, W)
    dinv = _sc_degree(er, zn)
    dinvb = jnp.broadcast_to(dinv[:, None], (_N, _D))
    g = _tc_scale(h, dinvb)
    gview = g.reshape(2 * _N, _DH)
    p = _sc_scatter(gview, er, z2)
    return _tc_ep(p, dinvb, b.reshape(1, _D))
